# baseline (speedup 1.0000x reference)
import functools

import jax
import jax.numpy as jnp
from jax import lax
from jax.experimental import pallas as pl
from jax.experimental.pallas import tpu as pltpu
from jax.experimental.pallas import tpu_sc as plsc

F32 = jnp.float32
NC = 2
NS = 16
NW = NC * NS
LN = 16
RB = 1000
KD = 80
KA = 128


def _sc_mesh():
    return plsc.VectorSubcoreMesh(core_axis_name="c", subcore_axis_name="s")


@functools.lru_cache(maxsize=None)
def _make_deg(N, NCH):
    RPS = N // NS

    def body(src_hbm, dst_hbm, out_hbm, srcb, dstb, obuf, zbuf, acc_o, acc_i,
             sem):
        c = lax.axis_index("c")
        s = lax.axis_index("s")
        wid = c * NS + s

        def fill_ones(i, carry):
            obuf[i, :] = jnp.ones((LN,), F32)
            return carry

        lax.fori_loop(0, KD, fill_ones, 0)

        def fill_zero(i, carry):
            zbuf[i, :] = jnp.zeros((LN,), F32)
            return carry

        lax.fori_loop(0, RPS, fill_zero, 0)
        pltpu.sync_copy(zbuf, acc_o.at[pl.ds(s * RPS, RPS)])
        pltpu.sync_copy(zbuf, acc_i.at[pl.ds(s * RPS, RPS)])
        plsc.subcore_barrier()

        pltpu.sync_copy(src_hbm.at[wid], srcb)
        pltpu.sync_copy(dst_hbm.at[wid], dstb)

        def step(t, carry):
            d = pltpu.async_copy(obuf, acc_o.at[srcb.at[t]], sem, add=True)
            pltpu.sync_copy(obuf, acc_i.at[dstb.at[t]], add=True)
            d.wait()
            return carry

        lax.fori_loop(0, NCH, step, 0)
        plsc.subcore_barrier()
        pltpu.sync_copy(acc_o.at[pl.ds(s * RPS, RPS)],
                        out_hbm.at[c, 0, pl.ds(s * RPS, RPS)])
        pltpu.sync_copy(acc_i.at[pl.ds(s * RPS, RPS)],
                        out_hbm.at[c, 1, pl.ds(s * RPS, RPS)])

    return pl.kernel(
        body,
        out_type=jax.ShapeDtypeStruct((NC, 2, N, LN), F32),
        mesh=_sc_mesh(),
        scratch_types=[
            pltpu.VMEM((NCH, KD), jnp.int32),
            pltpu.VMEM((NCH, KD), jnp.int32),
            pltpu.VMEM((KD, LN), F32),
            pltpu.VMEM((N // NS, LN), F32),
            pltpu.VMEM_SHARED((N, LN), F32),
            pltpu.VMEM_SHARED((N, LN), F32),
            pltpu.SemaphoreType.DMA,
        ],
        compiler_params=pltpu.CompilerParams(use_tc_tiling_on_sc=False),
    )


@functools.lru_cache(maxsize=None)
def _make_agg(N, H, NCH):
    HH = H // NC
    NACC = N + LN
    assert NACC % NS == 0
    RPZ = NACC // NS
    ZFULL, ZREM = RPZ // KA, RPZ % KA
    RPS = N // NS

    def body(g_hbm, src_hbm, dst_hbm, out_hbm, srcb, dstb, rows, rows2,
             acc, sem, sem2):
        c = lax.axis_index("c")
        s = lax.axis_index("s")

        def fill_zero(i, carry):
            for j in range(HH // LN):
                rows[i, pl.ds(j * LN, LN)] = jnp.zeros((LN,), F32)
            return carry

        lax.fori_loop(0, KA, fill_zero, 0)
        for kk in range(ZFULL):
            pltpu.sync_copy(rows, acc.at[pl.ds(s * RPZ + kk * KA, KA)])
        if ZREM:
            pltpu.sync_copy(rows.at[pl.ds(0, ZREM)],
                            acc.at[pl.ds(s * RPZ + ZFULL * KA, ZREM)])
        plsc.subcore_barrier()

        pltpu.sync_copy(src_hbm.at[c, s], srcb)
        pltpu.sync_copy(dst_hbm.at[s], dstb)

        def gstart(t, buf, gsem):
            pltpu.async_copy(g_hbm.at[srcb.at[t]], buf, gsem)

        def gwait(buf, gsem):
            pltpu.make_async_copy(g_hbm.at[pl.ds(0, KA)], buf, gsem).wait()

        gstart(0, rows, sem)

        def pair(i, carry):
            tA = 2 * i
            gstart(tA + 1, rows2, sem2)
            gwait(rows, sem)
            pltpu.sync_copy(rows, acc.at[dstb.at[tA]], add=True)

            @pl.when(tA + 2 < NCH)
            def _():
                gstart(tA + 2, rows, sem)

            gwait(rows2, sem2)
            pltpu.sync_copy(rows2, acc.at[dstb.at[tA + 1]], add=True)
            return carry

        assert NCH % 2 == 0
        lax.fori_loop(0, NCH // 2, pair, 0)
        plsc.subcore_barrier()
        pltpu.sync_copy(acc.at[pl.ds(s * RPS, RPS)],
                        out_hbm.at[pl.ds(s * RPS, RPS), pl.ds(c * HH, HH)])

    return pl.kernel(
        body,
        out_type=jax.ShapeDtypeStruct((N, H), F32),
        mesh=_sc_mesh(),
        scratch_types=[
            pltpu.VMEM((NCH, KA), jnp.int32),
            pltpu.VMEM((NCH, KA), jnp.int32),
            pltpu.VMEM((KA, HH), F32),
            pltpu.VMEM((KA, HH), F32),
            pltpu.VMEM_SHARED((NACC, HH), F32),
            pltpu.SemaphoreType.DMA,
            pltpu.SemaphoreType.DMA,
        ],
        compiler_params=pltpu.CompilerParams(use_tc_tiling_on_sc=False),
    )


def _elu(a):
    return jnp.where(a > 0, a, jnp.exp(jnp.minimum(a, 0.0)) - 1.0)


def _tc_matmul(x, W):
    N, D = x.shape
    H = W.shape[1]

    def body(x_ref, w_ref, o_ref):
        o_ref[...] = jnp.dot(x_ref[...], w_ref[...],
                             preferred_element_type=F32)

    return pl.pallas_call(
        body,
        grid=(N // RB,),
        in_specs=[
            pl.BlockSpec((RB, D), lambda i: (i, 0)),
            pl.BlockSpec((D, H), lambda i: (0, 0)),
        ],
        out_specs=pl.BlockSpec((RB, H), lambda i: (i, 0)),
        out_shape=jax.ShapeDtypeStruct((N, H), F32),
    )(x, W)


def _tc_first(dp4, m0):
    _, N, _ = dp4.shape
    H = m0.shape[1]
    HH = H // NC
    NB = N // RB

    def body(dp_ref, m_ref, g_ref, cs_ref, cd_ref):
        p = dp_ref[...]
        deg_o = (p[0] + p[2])[:, :1]
        deg_i = (p[1] + p[3])[:, :1]
        cs = lax.rsqrt(jnp.maximum(deg_o, 1.0))
        cd = lax.rsqrt(jnp.maximum(deg_i, 1.0))
        cs_b = jnp.broadcast_to(cs, (RB, HH))
        cd_ref[...] = jnp.broadcast_to(cd, (RB, H))
        m = m_ref[...]
        for c in range(NC):
            cs_ref[c] = cs_b
            g_ref[c] = m[:, c * HH:(c + 1) * HH] * cs_b

    return pl.pallas_call(
        body,
        grid=(NB,),
        in_specs=[
            pl.BlockSpec((4, RB, LN), lambda i: (0, i, 0)),
            pl.BlockSpec((RB, H), lambda i: (i, 0)),
        ],
        out_specs=[
            pl.BlockSpec((NC, RB, HH), lambda i: (0, i, 0)),
            pl.BlockSpec((NC, RB, HH), lambda i: (0, i, 0)),
            pl.BlockSpec((RB, H), lambda i: (i, 0)),
        ],
        out_shape=[
            jax.ShapeDtypeStruct((NC, N, HH), F32),
            jax.ShapeDtypeStruct((NC, N, HH), F32),
            jax.ShapeDtypeStruct((N, H), F32),
        ],
    )(dp4, m0)


def _tc_layer(p, cd_b, b, Wh, cs2):
    N, H = p.shape
    HH = Wh.shape[2]
    NB = N // RB

    def body(p_ref, cd_ref, b_ref, w_ref, cs_ref, g_ref):
        a = p_ref[...] * cd_ref[...] + b_ref[...]
        a = _elu(a)
        for c in range(NC):
            g_ref[c] = jnp.dot(a, w_ref[c],
                               preferred_element_type=F32) * cs_ref[c]

    return pl.pallas_call(
        body,
        grid=(NB,),
        in_specs=[
            pl.BlockSpec((RB, H), lambda i: (i, 0)),
            pl.BlockSpec((RB, H), lambda i: (i, 0)),
            pl.BlockSpec((1, H), lambda i: (0, 0)),
            pl.BlockSpec((NC, H, HH), lambda i: (0, 0, 0)),
            pl.BlockSpec((NC, RB, HH), lambda i: (0, i, 0)),
        ],
        out_specs=pl.BlockSpec((NC, RB, HH), lambda i: (0, i, 0)),
        out_shape=jax.ShapeDtypeStruct((NC, N, HH), F32),
    )(p, cd_b, b.reshape(1, H), Wh, cs2)


def _tc_final(p, cd_b, b, Wc, bc):
    N, H = p.shape
    C = Wc.shape[1]
    grid = N // RB

    def body(p_ref, cd_ref, b_ref, wc_ref, bc_ref, out_ref, acc_ref):
        i = pl.program_id(0)
        a = p_ref[...] * cd_ref[...] + b_ref[...]
        a = _elu(a)
        blk = jnp.sum(a, axis=0, keepdims=True)

        @pl.when(i == 0)
        def _init():
            acc_ref[...] = jnp.zeros_like(acc_ref)

        acc_ref[...] += blk

        @pl.when(i == grid - 1)
        def _fin():
            m = acc_ref[...] * (1.0 / N)
            out_ref[...] = jnp.dot(m, wc_ref[...],
                                   preferred_element_type=F32) + bc_ref[...]

    return pl.pallas_call(
        body,
        grid=(grid,),
        in_specs=[
            pl.BlockSpec((RB, H), lambda i: (i, 0)),
            pl.BlockSpec((RB, H), lambda i: (i, 0)),
            pl.BlockSpec((1, H), lambda i: (0, 0)),
            pl.BlockSpec((H, C), lambda i: (0, 0)),
            pl.BlockSpec((1, C), lambda i: (0, 0)),
        ],
        out_specs=pl.BlockSpec((1, C), lambda i: (0, 0)),
        out_shape=jax.ShapeDtypeStruct((1, C), F32),
        scratch_shapes=[pltpu.VMEM((1, H), F32)],
    )(p, cd_b, b.reshape(1, H), Wc, bc.reshape(1, C))


def kernel(x, edge_index, W0, b0, W1, b1, W2, b2, Wc, bc):
    N, D = x.shape
    H = W0.shape[1]
    E = edge_index.shape[1]
    assert E % NW == 0 and N % NS == 0 and N % RB == 0

    PWD = E // NW
    assert PWD % KD == 0
    src_d = edge_index[0].reshape(NW, PWD // KD, KD)
    dst_d = edge_index[1].reshape(NW, PWD // KD, KD)

    PWA = E // NS
    PWP = ((PWA + KA - 1) // KA) * KA
    NCH = PWP // KA
    if NCH % 2:
        PWP += KA
        NCH += 1
    pad = PWP - PWA
    src_a = jnp.pad(edge_index[0].reshape(NS, PWA), ((0, 0), (0, pad)))
    dst_a = jnp.pad(edge_index[1].reshape(NS, PWA), ((0, 0), (0, pad)),
                    constant_values=N)
    src_a = jnp.stack([src_a, src_a + N]).reshape(NC, NS, NCH, KA)
    dst_a = dst_a.reshape(NS, NCH, KA)

    HH = H // NC
    W1h = jnp.stack([W1[:, c * HH:(c + 1) * HH] for c in range(NC)])
    W2h = jnp.stack([W2[:, c * HH:(c + 1) * HH] for c in range(NC)])

    dp = _make_deg(N, PWD // KD)(src_d, dst_d)
    m0 = _tc_matmul(x, W0)
    g2, cs2, cd_b = _tc_first(dp.reshape(NC * 2, N, LN), m0)

    agg = _make_agg(N, H, NCH)
    p = agg(g2.reshape(NC * N, HH), src_a, dst_a)
    g2 = _tc_layer(p, cd_b, b0, W1h, cs2)
    p = agg(g2.reshape(NC * N, HH), src_a, dst_a)
    g2 = _tc_layer(p, cd_b, b1, W2h, cs2)
    p = agg(g2.reshape(NC * N, HH), src_a, dst_a)
    return _tc_final(p, cd_b, b2, Wc, bc)
